# baseline (speedup 1.0000x reference)
import functools

import jax
import jax.numpy as jnp
from jax import lax
from jax.experimental import pallas as pl
from jax.experimental.pallas import tpu as pltpu
from jax.experimental.pallas import tpu_sc as plsc


def _make_sc_gather(B0, S, V, D, n_workers, nc):
    L = 128
    B = B0 * S
    rows_per_w = B0 // n_workers
    CR = 8
    n_chunks = rows_per_w // CR
    mesh = plsc.VectorSubcoreMesh(core_axis_name="c", subcore_axis_name="s")

    @functools.partial(
        pl.kernel,
        mesh=mesh,
        compiler_params=pltpu.CompilerParams(use_tc_tiling_on_sc=False),
        out_type=jax.ShapeDtypeStruct((B, L), jnp.float32),
        scratch_types=[
            pltpu.VMEM((rows_per_w, S), jnp.int32),
            pltpu.VMEM((CR * S, L), jnp.float32),
            pltpu.VMEM((CR * S, L), jnp.float32),
            pltpu.SemaphoreType.DMA,
            pltpu.SemaphoreType.DMA,
            pltpu.SemaphoreType.DMA,
            pltpu.SemaphoreType.DMA,
        ],
    )
    def k(ids_hbm, table_hbm, mid_hbm, idx_all, rows0, rows1,
          semg0, semg1, sems0, sems1):
        wid = lax.axis_index("s") * nc + lax.axis_index("c")
        base = wid * rows_per_w
        pltpu.sync_copy(ids_hbm.at[pl.ds(base, rows_per_w)], idx_all)

        bufs = (rows0, rows1)
        gsems = (semg0, semg1)
        ssems = (sems0, sems1)

        def start_gathers(i):
            buf = bufs[i % 2]
            return [
                pltpu.async_copy(
                    table_hbm.at[idx_all.at[i * CR + j]],
                    buf.at[pl.ds(j * S, S)],
                    gsems[i % 2],
                )
                for j in range(CR)
            ]

        def start_store(i):
            return pltpu.async_copy(
                bufs[i % 2],
                mid_hbm.at[pl.ds((base + i * CR) * S, CR * S)],
                ssems[i % 2],
            )

        gcps = [None] * n_chunks
        scps = [None] * n_chunks
        gcps[0] = start_gathers(0)
        for i in range(n_chunks):
            for cp in gcps[i]:
                cp.wait()
            if i >= 1:
                scps[i - 1].wait()
            if i + 1 < n_chunks:
                gcps[i + 1] = start_gathers(i + 1)
            scps[i] = start_store(i)
        scps[n_chunks - 1].wait()

    return k


def _make_tc_finish(B0, S, D):
    L = 128
    RB = 256
    grid = B0 // RB

    def body(mid_ref, out_ref):
        y = mid_ref[...]
        z = y[:, :D].reshape(RB, S, D)
        out_ref[...] = jnp.transpose(z, (1, 2, 0))

    return pl.pallas_call(
        body,
        grid=(grid,),
        in_specs=[pl.BlockSpec((RB * S, L), lambda i: (i, 0))],
        out_specs=pl.BlockSpec((S, D, RB), lambda i: (0, 0, i)),
        out_shape=jax.ShapeDtypeStruct((S, D, B0), jnp.float32),
    )


def kernel(entity_ids, table):
    B0, S = entity_ids.shape
    V, D = table.shape
    info = plsc.get_sparse_core_info()
    n_workers = info.num_cores * info.num_subcores
    ids = entity_ids.astype(jnp.int32)
    tpad = jnp.pad(table, ((0, 0), (0, 128 - D)))
    mid = _make_sc_gather(B0, S, V, D, n_workers, info.num_cores)(ids, tpad)
    return mid[:, :D].reshape(B0, S, D)


# speedup vs baseline: 1.2189x; 1.2189x over previous
import functools

import jax
import jax.numpy as jnp
from jax import lax
from jax.experimental import pallas as pl
from jax.experimental.pallas import tpu as pltpu
from jax.experimental.pallas import tpu_sc as plsc


def _make_gather(V, D, B, n_workers, nc):
    b_per_w = B // n_workers
    C = 800
    n_chunks = b_per_w // C
    mesh = plsc.VectorSubcoreMesh(core_axis_name="c", subcore_axis_name="s")

    @functools.partial(
        pl.kernel,
        mesh=mesh,
        compiler_params=pltpu.CompilerParams(use_tc_tiling_on_sc=False),
        out_type=jax.ShapeDtypeStruct((B, D), jnp.float32),
        scratch_types=[
            pltpu.VMEM((b_per_w,), jnp.int32),
            pltpu.VMEM((C, D), jnp.float32),
            pltpu.VMEM((C, D), jnp.float32),
            pltpu.SemaphoreType.DMA,
            pltpu.SemaphoreType.DMA,
        ],
    )
    def k(ids_hbm, table_hbm, out_hbm, idx_all, rows0, rows1, sem0, sem1):
        wid = lax.axis_index("s") * nc + lax.axis_index("c")
        base = wid * b_per_w
        pltpu.sync_copy(ids_hbm.at[pl.ds(base, b_per_w)], idx_all)
        bufs = (rows0, rows1)
        sems = (sem0, sem1)

        def start(i):
            return pltpu.async_copy(
                table_hbm.at[idx_all.at[pl.ds(i * C, C)]], bufs[i % 2], sems[i % 2]
            )

        cps = [None] * n_chunks
        cps[0] = start(0)
        for i in range(n_chunks):
            cps[i].wait()
            if i + 1 < n_chunks:
                cps[i + 1] = start(i + 1)
            pltpu.sync_copy(bufs[i % 2], out_hbm.at[pl.ds(base + i * C, C)])

    return k


def kernel(entity_ids, table):
    B0, S = entity_ids.shape
    V, D = table.shape
    B = B0 * S
    info = plsc.get_sparse_core_info()
    n_workers = info.num_cores * info.num_subcores
    ids = entity_ids.reshape(B).astype(jnp.int32)
    out = _make_gather(V, D, B, n_workers, info.num_cores)(ids, table)
    return out.reshape(B0, S, D)
